# baseline (speedup 1.0000x reference)
import functools

import jax
import jax.numpy as jnp
from jax import lax
from jax.experimental import pallas as pl
from jax.experimental.pallas import tpu as pltpu
from jax.experimental.pallas import tpu_sc as plsc

N = 10000
E = 320000
D = 128
NPAD = 10112
TILES = 32
ET = E // TILES
B = 64
CH = 160
CPB = 16
RPT = NPAD // 16

_f32 = jnp.float32
_i32 = jnp.int32


def _make_sc_agg(with_cnt):
    mesh = plsc.VectorSubcoreMesh(core_axis_name="c", subcore_axis_name="s",
                                  num_cores=2, num_subcores=16)
    CNTW = NPAD * 16
    out_type = [jax.ShapeDtypeStruct((2 * NPAD, D), _f32)]
    if with_cnt:
        out_type.append(jax.ShapeDtypeStruct((2 * CNTW,), _f32))
    NG = B // 16

    def body(*refs):
        if with_cnt:
            (h_hbm, srcp, dstp, efp, znd, zcnt, out_agg, out_cnt,
             agg_sh, cnt_sh, sidx0, sidx1, didx0, didx1, efv0, efv1,
             rows0, rows1, rows2, sdix0, sdix1, sdix2,
             fidx, ones_v,
             gsem0, gsem1, gsem2, isem0, isem1,
             ssem0, ssem1, ssem2) = refs
        else:
            (h_hbm, srcp, dstp, znd, out_agg,
             agg_sh, sidx0, sidx1, didx0, didx1,
             rows0, rows1, rows2, sdix0, sdix1, sdix2,
             gsem0, gsem1, gsem2, isem0, isem1,
             ssem0, ssem1, ssem2) = refs
            cnt_sh = fidx = ones_v = out_cnt = efp = zcnt = None
            efv0 = efv1 = None
        rows = [rows0, rows1, rows2]
        gsem = [gsem0, gsem1, gsem2]
        ssem = [ssem0, ssem1, ssem2]
        sdix = [sdix0, sdix1, sdix2]
        sidx = [sidx0, sidx1]
        didx = [didx0, didx1]
        efv = [efv0, efv1]
        isem = [isem0, isem1]

        c = lax.axis_index("c")
        s = lax.axis_index("s")
        t = c * 16 + s

        pltpu.sync_copy(znd.at[pl.ds(s * RPT, RPT)], agg_sh.at[pl.ds(s * RPT, RPT)])
        if with_cnt:
            pltpu.sync_copy(zcnt.at[pl.ds(s * RPT * 16, RPT * 16)],
                            cnt_sh.at[pl.ds(s * RPT * 16, RPT * 16)])
            for g in range(NG):
                ones_v[pl.ds(g * 16, 16)] = jnp.ones((16,), _f32)
        plsc.subcore_barrier()

        def issue_idx(bank, bi):
            pltpu.async_copy(srcp.at[t, pl.ds(bi * CPB, CPB)], sidx[bank], isem[bank])
            pltpu.async_copy(dstp.at[t, pl.ds(bi * CPB, CPB)], didx[bank], isem[bank])
            if with_cnt:
                pltpu.async_copy(efp.at[t, pl.ds(bi * CPB, CPB)], efv[bank], isem[bank])

        def wait_idx(bank, bi):
            pltpu.make_async_copy(srcp.at[t, pl.ds(bi * CPB, CPB)], sidx[bank],
                                  isem[bank]).wait()
            pltpu.make_async_copy(dstp.at[t, pl.ds(bi * CPB, CPB)], didx[bank],
                                  isem[bank]).wait()
            if with_cnt:
                pltpu.make_async_copy(efp.at[t, pl.ds(bi * CPB, CPB)], efv[bank],
                                      isem[bank]).wait()

        def process(bank, first=False):
            sb, db, eb = sidx[bank], didx[bank], (efv[bank] if with_cnt else None)
            cps = [None, None, None]
            for k in range(2):
                if not first:
                    pltpu.make_async_copy(rows[k], agg_sh.at[sdix[k]],
                                          ssem[k]).wait()
                cps[k] = pltpu.async_copy(h_hbm.at[sb.at[k]], rows[k], gsem[k])
            for j in range(CPB):
                b = j % 3
                if j + 2 < CPB:
                    bn = (j + 2) % 3
                    if not (first and j == 0):
                        pltpu.make_async_copy(
                            rows[bn], agg_sh.at[sdix[bn]], ssem[bn]).wait()
                    cps[bn] = pltpu.async_copy(
                        h_hbm.at[sb.at[j + 2]], rows[bn], gsem[bn])
                if with_cnt:
                    for g in range(NG):
                        dvec = db.at[j][pl.ds(g * 16, 16)]
                        evec = eb.at[j][pl.ds(g * 16, 16)]
                        fidx[pl.ds(g * 16, 16)] = dvec * 16 + evec
                for g in range(NG):
                    sdix[b][pl.ds(g * 16, 16)] = db.at[j][pl.ds(g * 16, 16)]
                cps[b].wait()
                pltpu.async_copy(rows[b], agg_sh.at[sdix[b]], ssem[b], add=True)
                if with_cnt:
                    pltpu.sync_copy(ones_v, cnt_sh.at[fidx], add=True)

        NBK = CH // CPB
        issue_idx(0, 0)

        wait_idx(0, 0)
        issue_idx(1, 1)
        process(0, first=True)
        issue_idx(0, 2)
        wait_idx(1, 1)
        process(1)

        def sblk(i, carry):
            wait_idx(0, 2 * i)
            issue_idx(1, 2 * i + 1)
            process(0)

            @pl.when(i < NBK // 2 - 1)
            def _():
                issue_idx(0, 2 * i + 2)

            wait_idx(1, 2 * i + 1)
            process(1)
            return carry

        lax.fori_loop(1, NBK // 2, sblk, 0)

        for k in range(3):
            pltpu.make_async_copy(rows[k], agg_sh.at[sdix[k]], ssem[k]).wait()
        plsc.subcore_barrier()

        pltpu.sync_copy(agg_sh.at[pl.ds(s * RPT, RPT)],
                        out_agg.at[pl.ds(c * NPAD + s * RPT, RPT)])
        if with_cnt:
            pltpu.sync_copy(cnt_sh.at[pl.ds(s * RPT * 16, RPT * 16)],
                            out_cnt.at[pl.ds(c * CNTW + s * RPT * 16, RPT * 16)])

    scratch = [pltpu.VMEM_SHARED((NPAD, D), _f32)]
    if with_cnt:
        scratch.append(pltpu.VMEM_SHARED((CNTW,), _f32))
    scratch += [
        pltpu.VMEM((CPB, B), _i32),
        pltpu.VMEM((CPB, B), _i32),
        pltpu.VMEM((CPB, B), _i32),
        pltpu.VMEM((CPB, B), _i32),
    ]
    if with_cnt:
        scratch.append(pltpu.VMEM((CPB, B), _i32))
        scratch.append(pltpu.VMEM((CPB, B), _i32))
    for _ in range(3):
        scratch.append(pltpu.VMEM((B, D), _f32))
    for _ in range(3):
        scratch.append(pltpu.VMEM((B,), _i32))
    if with_cnt:
        scratch.append(pltpu.VMEM((B,), _i32))
        scratch.append(pltpu.VMEM((B,), _f32))
    for _ in range(8):
        scratch.append(pltpu.SemaphoreType.DMA)

    return pl.kernel(body, out_type=out_type, mesh=mesh, scratch_types=scratch)


_make_sc_agg = functools.lru_cache(maxsize=None)(_make_sc_agg)


def _embed_body(nf_ref, at_ref, out_ref):
    cols = lax.broadcasted_iota(_i32, (NPAD, 128), 1)
    oh = (cols == nf_ref[...]).astype(_f32)
    out_ref[...] = jnp.dot(oh, at_ref[...], preferred_element_type=_f32)


_embed = pl.pallas_call(
    _embed_body, out_shape=jax.ShapeDtypeStruct((NPAD, D), _f32))


def _gn(x, xs, g, b):
    m = jnp.mean(xs, axis=0, keepdims=True)
    v = jnp.mean((xs - m) ** 2, axis=0, keepdims=True)
    return (x - m) * lax.rsqrt(v + 1e-5) * g + b


def _mlp_body(last, aggs, cnts, btp, h, Wi, bi, gg, gb, Wo, bo, og, ob, *rest):
    if last:
        pW, pb, out = rest
    else:
        (out,) = rest
    agg = aggs[0] + aggs[1] + (cnts[0] + cnts[1]) @ btp[...]
    x = h[...]
    r = (agg + x) @ Wi[...] + bi[...]
    r = _gn(r, r[:N], gg[...], gb[...])
    r = jnp.maximum(r, 0.0) @ Wo[...] + bo[...]
    r = _gn(r, r[:N], og[...], ob[...])
    if not last:
        r = jnp.maximum(r, 0.0)
    hn = r + x
    if last:
        gmean = jnp.mean(hn[:N], axis=0, keepdims=True)
        out[...] = gmean @ pW[...] + pb[...]
    else:
        out[...] = hn


_mlp0 = pl.pallas_call(
    functools.partial(_mlp_body, False),
    out_shape=jax.ShapeDtypeStruct((NPAD, D), _f32))
_mlp1 = pl.pallas_call(
    functools.partial(_mlp_body, True),
    out_shape=jax.ShapeDtypeStruct((1, 128), _f32))


def kernel(nfeat, efeat, edge_index, atom_table,
           bond_table_0, W_in_0, b_in_0, gn_h_g_0, gn_h_b_0, W_out_0, b_out_0,
           gn_o_g_0, gn_o_b_0,
           bond_table_1, W_in_1, b_in_1, gn_h_g_1, gn_h_b_1, W_out_1, b_out_1,
           gn_o_g_1, gn_o_b_1,
           predict_W, predict_b):
    pad_len = CH * B - ET

    trash = N + (jnp.arange(pad_len, dtype=_i32) % (NPAD - N))
    tpad = jnp.broadcast_to(trash, (TILES, pad_len))
    zpad = jnp.zeros((TILES, pad_len), _i32)

    def pad_idx(a, pad):
        a2 = a.reshape(TILES, ET)
        return jnp.concatenate([a2, pad], axis=1).reshape(TILES, CH, B)

    srcp = pad_idx(edge_index[0], tpad)
    dstp = pad_idx(edge_index[1], tpad)
    efp = pad_idx(efeat, zpad)

    znd = jnp.zeros((NPAD, D), _f32)
    zcnt = jnp.zeros((NPAD * 16,), _f32)

    nf2 = jnp.concatenate([nfeat, jnp.zeros((NPAD - N,), _i32)]).reshape(NPAD, 1)
    at_pad = jnp.concatenate(
        [atom_table, jnp.zeros((128 - atom_table.shape[0], D), _f32)])
    bt0 = jnp.concatenate([bond_table_0, jnp.zeros((11, D), _f32)])
    bt1 = jnp.concatenate([bond_table_1, jnp.zeros((11, D), _f32)])

    h0 = _embed(nf2, at_pad)

    agg0, cnt = _make_sc_agg(True)(h0, srcp, dstp, efp, znd, zcnt)
    agg0 = agg0.reshape(2, NPAD, D)
    cnt = cnt.reshape(2, NPAD, 16)

    h1 = _mlp0(agg0, cnt, bt0, h0,
               W_in_0, b_in_0.reshape(1, -1), gn_h_g_0.reshape(1, -1),
               gn_h_b_0.reshape(1, -1), W_out_0, b_out_0.reshape(1, -1),
               gn_o_g_0.reshape(1, -1), gn_o_b_0.reshape(1, -1))

    res1 = _make_sc_agg(False)(h1, srcp, dstp, znd)
    agg1 = res1[0] if isinstance(res1, (tuple, list)) else res1
    agg1 = agg1.reshape(2, NPAD, D)

    pre = _mlp1(agg1, cnt, bt1, h1,
                W_in_1, b_in_1.reshape(1, -1), gn_h_g_1.reshape(1, -1),
                gn_h_b_1.reshape(1, -1), W_out_1, b_out_1.reshape(1, -1),
                gn_o_g_1.reshape(1, -1), gn_o_b_1.reshape(1, -1),
                predict_W, predict_b.reshape(1, -1))
    return pre


# speedup vs baseline: 1.0200x; 1.0200x over previous
import functools

import jax
import jax.numpy as jnp
from jax import lax
from jax.experimental import pallas as pl
from jax.experimental.pallas import tpu as pltpu
from jax.experimental.pallas import tpu_sc as plsc

N = 10000
E = 320000
D = 128
NPAD = 10112
TILES = 32
ET = E // TILES
B = 128
CH = (ET + B - 1) // B + 1
RPT = NPAD // 16

_f32 = jnp.float32
_i32 = jnp.int32


def _make_sc_agg(with_cnt):
    mesh = plsc.VectorSubcoreMesh(core_axis_name="c", subcore_axis_name="s",
                                  num_cores=2, num_subcores=16)
    CNTW = NPAD * 16
    out_type = [jax.ShapeDtypeStruct((2 * NPAD, D), _f32)]
    if with_cnt:
        out_type.append(jax.ShapeDtypeStruct((2 * CNTW,), _f32))

    def body(*refs):
        if with_cnt:
            (h_hbm, srcp, dstp, efp, znd, zcnt, out_agg, out_cnt,
             agg_sh, cnt_sh, sidx0, sidx1, didx0, didx1, efv0, efv1,
             rows0, rows1, sdix0, sdix1, fidx, ones_v,
             gsem0, gsem1, isem0, isem1, ssem0, ssem1) = refs
        else:
            (h_hbm, srcp, dstp, znd, out_agg,
             agg_sh, sidx0, sidx1, didx0, didx1,
             rows0, rows1, sdix0, sdix1,
             gsem0, gsem1, isem0, isem1, ssem0, ssem1) = refs
            cnt_sh = fidx = ones_v = out_cnt = efp = zcnt = None
            efv0 = efv1 = None
        rows = [rows0, rows1]
        gsem = [gsem0, gsem1]
        sidx = [sidx0, sidx1]
        didx = [didx0, didx1]
        efv = [efv0, efv1]
        isem = [isem0, isem1]
        sdix = [sdix0, sdix1]
        ssem = [ssem0, ssem1]

        c = lax.axis_index("c")
        s = lax.axis_index("s")
        t = c * 16 + s

        pltpu.sync_copy(znd.at[pl.ds(s * RPT, RPT)], agg_sh.at[pl.ds(s * RPT, RPT)])
        if with_cnt:
            pltpu.sync_copy(zcnt.at[pl.ds(s * RPT * 16, RPT * 16)],
                            cnt_sh.at[pl.ds(s * RPT * 16, RPT * 16)])
            for g in range(8):
                ones_v[pl.ds(g * 16, 16)] = jnp.ones((16,), _f32)
        plsc.subcore_barrier()

        def issue_idx(bank, bi):
            pltpu.async_copy(srcp.at[t, pl.ds(bi * 8, 8)], sidx[bank], isem[bank])
            pltpu.async_copy(dstp.at[t, pl.ds(bi * 8, 8)], didx[bank], isem[bank])
            if with_cnt:
                pltpu.async_copy(efp.at[t, pl.ds(bi * 8, 8)], efv[bank], isem[bank])

        def wait_idx(bank, bi):
            pltpu.make_async_copy(srcp.at[t, pl.ds(bi * 8, 8)], sidx[bank],
                                  isem[bank]).wait()
            pltpu.make_async_copy(dstp.at[t, pl.ds(bi * 8, 8)], didx[bank],
                                  isem[bank]).wait()
            if with_cnt:
                pltpu.make_async_copy(efp.at[t, pl.ds(bi * 8, 8)], efv[bank],
                                      isem[bank]).wait()

        def process(bank, first=False):
            sb, db, eb = sidx[bank], didx[bank], (efv[bank] if with_cnt else None)
            if not first:
                pltpu.make_async_copy(rows[0], agg_sh.at[sdix[0]],
                                      ssem[0]).wait()
            cps = [None, None]
            cps[0] = pltpu.async_copy(h_hbm.at[sb.at[0]], rows[0], gsem[0])
            for j in range(8):
                b = j % 2
                bn = (j + 1) % 2
                if j < 7:
                    if not (first and j == 0):
                        pltpu.make_async_copy(
                            rows[bn], agg_sh.at[sdix[bn]], ssem[bn]).wait()
                    cps[bn] = pltpu.async_copy(
                        h_hbm.at[sb.at[j + 1]], rows[bn], gsem[bn])
                if with_cnt:
                    for g in range(8):
                        dvec = db.at[j][pl.ds(g * 16, 16)]
                        evec = eb.at[j][pl.ds(g * 16, 16)]
                        fidx[pl.ds(g * 16, 16)] = dvec * 16 + evec
                for g in range(8):
                    sdix[b][pl.ds(g * 16, 16)] = db.at[j][pl.ds(g * 16, 16)]
                cps[b].wait()
                pltpu.async_copy(rows[b], agg_sh.at[sdix[b]], ssem[b], add=True)
                if with_cnt:
                    pltpu.sync_copy(ones_v, cnt_sh.at[fidx], add=True)

        issue_idx(0, 0)

        wait_idx(0, 0)
        issue_idx(1, 1)
        process(0, first=True)
        issue_idx(0, 2)
        wait_idx(1, 1)
        process(1)

        def sblk(i, carry):
            wait_idx(0, 2 * i)
            issue_idx(1, 2 * i + 1)
            process(0)

            @pl.when(i < CH // 16 - 1)
            def _():
                issue_idx(0, 2 * i + 2)

            wait_idx(1, 2 * i + 1)
            process(1)
            return carry

        lax.fori_loop(1, CH // 16, sblk, 0)

        pltpu.make_async_copy(rows[0], agg_sh.at[sdix[0]], ssem[0]).wait()
        pltpu.make_async_copy(rows[1], agg_sh.at[sdix[1]], ssem[1]).wait()
        plsc.subcore_barrier()

        pltpu.sync_copy(agg_sh.at[pl.ds(s * RPT, RPT)],
                        out_agg.at[pl.ds(c * NPAD + s * RPT, RPT)])
        if with_cnt:
            pltpu.sync_copy(cnt_sh.at[pl.ds(s * RPT * 16, RPT * 16)],
                            out_cnt.at[pl.ds(c * CNTW + s * RPT * 16, RPT * 16)])

    scratch = [pltpu.VMEM_SHARED((NPAD, D), _f32)]
    if with_cnt:
        scratch.append(pltpu.VMEM_SHARED((CNTW,), _f32))
    scratch += [
        pltpu.VMEM((8, B), _i32),
        pltpu.VMEM((8, B), _i32),
        pltpu.VMEM((8, B), _i32),
        pltpu.VMEM((8, B), _i32),
    ]
    if with_cnt:
        scratch.append(pltpu.VMEM((8, B), _i32))
        scratch.append(pltpu.VMEM((8, B), _i32))
    scratch.append(pltpu.VMEM((B, D), _f32))
    scratch.append(pltpu.VMEM((B, D), _f32))
    scratch.append(pltpu.VMEM((B,), _i32))
    scratch.append(pltpu.VMEM((B,), _i32))
    if with_cnt:
        scratch.append(pltpu.VMEM((B,), _i32))
        scratch.append(pltpu.VMEM((B,), _f32))
    for _ in range(6):
        scratch.append(pltpu.SemaphoreType.DMA)

    return pl.kernel(body, out_type=out_type, mesh=mesh, scratch_types=scratch)


_make_sc_agg = functools.lru_cache(maxsize=None)(_make_sc_agg)


def _embed_body(nf_ref, at_ref, out_ref):
    cols = lax.broadcasted_iota(_i32, (NPAD, 128), 1)
    oh = (cols == nf_ref[...]).astype(_f32)
    out_ref[...] = jnp.dot(oh, at_ref[...], preferred_element_type=_f32)


_embed = pl.pallas_call(
    _embed_body, out_shape=jax.ShapeDtypeStruct((NPAD, D), _f32))


def _gn(x, xs, g, b):
    m = jnp.mean(xs, axis=0, keepdims=True)
    v = jnp.mean((xs - m) ** 2, axis=0, keepdims=True)
    return (x - m) * lax.rsqrt(v + 1e-5) * g + b


def _mlp_body(last, aggs, cnts, btp, h, Wi, bi, gg, gb, Wo, bo, og, ob, *rest):
    if last:
        pW, pb, out = rest
    else:
        (out,) = rest
    agg = aggs[0] + aggs[1] + (cnts[0] + cnts[1]) @ btp[...]
    x = h[...]
    r = (agg + x) @ Wi[...] + bi[...]
    r = _gn(r, r[:N], gg[...], gb[...])
    r = jnp.maximum(r, 0.0) @ Wo[...] + bo[...]
    r = _gn(r, r[:N], og[...], ob[...])
    if not last:
        r = jnp.maximum(r, 0.0)
    hn = r + x
    if last:
        gmean = jnp.mean(hn[:N], axis=0, keepdims=True)
        out[...] = gmean @ pW[...] + pb[...]
    else:
        out[...] = hn


_mlp0 = pl.pallas_call(
    functools.partial(_mlp_body, False),
    out_shape=jax.ShapeDtypeStruct((NPAD, D), _f32))
_mlp1 = pl.pallas_call(
    functools.partial(_mlp_body, True),
    out_shape=jax.ShapeDtypeStruct((1, 128), _f32))


def kernel(nfeat, efeat, edge_index, atom_table,
           bond_table_0, W_in_0, b_in_0, gn_h_g_0, gn_h_b_0, W_out_0, b_out_0,
           gn_o_g_0, gn_o_b_0,
           bond_table_1, W_in_1, b_in_1, gn_h_g_1, gn_h_b_1, W_out_1, b_out_1,
           gn_o_g_1, gn_o_b_1,
           predict_W, predict_b):
    pad_len = CH * B - ET

    trash = N + (jnp.arange(pad_len, dtype=_i32) % (NPAD - N))
    tpad = jnp.broadcast_to(trash, (TILES, pad_len))
    zpad = jnp.zeros((TILES, pad_len), _i32)

    def pad_idx(a, pad):
        a2 = a.reshape(TILES, ET)
        return jnp.concatenate([a2, pad], axis=1).reshape(TILES, CH, B)

    srcp = pad_idx(edge_index[0], tpad)
    dstp = pad_idx(edge_index[1], tpad)
    efp = pad_idx(efeat, zpad)

    znd = jnp.zeros((NPAD, D), _f32)
    zcnt = jnp.zeros((NPAD * 16,), _f32)

    nf2 = jnp.concatenate([nfeat, jnp.zeros((NPAD - N,), _i32)]).reshape(NPAD, 1)
    at_pad = jnp.concatenate(
        [atom_table, jnp.zeros((128 - atom_table.shape[0], D), _f32)])
    bt0 = jnp.concatenate([bond_table_0, jnp.zeros((11, D), _f32)])
    bt1 = jnp.concatenate([bond_table_1, jnp.zeros((11, D), _f32)])

    h0 = _embed(nf2, at_pad)

    agg0, cnt = _make_sc_agg(True)(h0, srcp, dstp, efp, znd, zcnt)
    agg0 = agg0.reshape(2, NPAD, D)
    cnt = cnt.reshape(2, NPAD, 16)

    h1 = _mlp0(agg0, cnt, bt0, h0,
               W_in_0, b_in_0.reshape(1, -1), gn_h_g_0.reshape(1, -1),
               gn_h_b_0.reshape(1, -1), W_out_0, b_out_0.reshape(1, -1),
               gn_o_g_0.reshape(1, -1), gn_o_b_0.reshape(1, -1))

    res1 = _make_sc_agg(False)(h1, srcp, dstp, znd)
    agg1 = res1[0] if isinstance(res1, (tuple, list)) else res1
    agg1 = agg1.reshape(2, NPAD, D)

    pre = _mlp1(agg1, cnt, bt1, h1,
                W_in_1, b_in_1.reshape(1, -1), gn_h_g_1.reshape(1, -1),
                gn_h_b_1.reshape(1, -1), W_out_1, b_out_1.reshape(1, -1),
                gn_o_g_1.reshape(1, -1), gn_o_b_1.reshape(1, -1),
                predict_W, predict_b.reshape(1, -1))
    return pre
